# Initial kernel scaffold; baseline (speedup 1.0000x reference)
import jax
import jax.numpy as jnp
from jax.experimental import pallas as pl


def kernel(key_ids, tempo_values, time_sig_ids, key_table, time_sig_table, W1, b1, W2, b2, Wp, bp):
    raise NotImplementedError("write your pallas kernel here")


# speedup vs baseline: 8.5941x; 8.5941x over previous
import jax
import jax.numpy as jnp
from jax import lax
from jax.experimental import pallas as pl

_MIN_TEMPO = 90.0
_MAX_TEMPO = 140.0


def _body(kid_ref, tv_ref, sid_ref, kt_ref, st_ref, w1_ref, b1_ref,
          w2_ref, b2_ref, wp_ref, bp_ref, out_ref):
    blk = out_ref.shape[0]
    nkey = kt_ref.shape[0]
    nsig = st_ref.shape[0]

    kid = kid_ref[0, 0, :]
    sid = sid_ref[0, 0, :]
    tv = tv_ref[0, 0, :]

    ohk = (kid[:, None] == lax.broadcasted_iota(jnp.int32, (blk, nkey), 1)
           ).astype(jnp.float32)
    ohs = (sid[:, None] == lax.broadcasted_iota(jnp.int32, (blk, nsig), 1)
           ).astype(jnp.float32)
    key_emb = jnp.dot(ohk, kt_ref[...], preferred_element_type=jnp.float32)
    sig_emb = jnp.dot(ohs, st_ref[...], preferred_element_type=jnp.float32)

    tn = jnp.where(tv > 0, (tv - _MIN_TEMPO) / (_MAX_TEMPO - _MIN_TEMPO),
                   jnp.zeros_like(tv))
    h = jnp.maximum(tn[:, None] * w1_ref[...] + b1_ref[...], 0.0)
    tempo_emb = jnp.dot(h, w2_ref[...],
                        preferred_element_type=jnp.float32) + b2_ref[...]

    combined = jnp.concatenate([key_emb, tempo_emb, sig_emb], axis=1)
    out_ref[...] = jnp.dot(combined, wp_ref[...],
                           preferred_element_type=jnp.float32) + bp_ref[...]


def kernel(key_ids, tempo_values, time_sig_ids, key_table, time_sig_table,
           W1, b1, W2, b2, Wp, bp):
    B = key_ids.shape[0]
    H = Wp.shape[1]
    BLK = 2048 if B % 2048 == 0 else B
    NB = B // BLK

    kid3 = key_ids.astype(jnp.int32).reshape(NB, 1, BLK)
    tv3 = tempo_values.reshape(NB, 1, BLK)
    sid3 = time_sig_ids.astype(jnp.int32).reshape(NB, 1, BLK)
    b1r = b1.reshape(1, -1)
    b2r = b2.reshape(1, -1)
    bpr = bp.reshape(1, -1)

    def blk_spec(shape):
        nd = len(shape)
        return pl.BlockSpec(shape, lambda i, _nd=nd: (0,) * _nd)

    return pl.pallas_call(
        _body,
        grid=(NB,),
        in_specs=[
            pl.BlockSpec((1, 1, BLK), lambda i: (i, 0, 0)),
            pl.BlockSpec((1, 1, BLK), lambda i: (i, 0, 0)),
            pl.BlockSpec((1, 1, BLK), lambda i: (i, 0, 0)),
            blk_spec(key_table.shape),
            blk_spec(time_sig_table.shape),
            blk_spec(W1.shape),
            blk_spec(b1r.shape),
            blk_spec(W2.shape),
            blk_spec(b2r.shape),
            blk_spec(Wp.shape),
            blk_spec(bpr.shape),
        ],
        out_specs=pl.BlockSpec((BLK, H), lambda i: (i, 0)),
        out_shape=jax.ShapeDtypeStruct((B, H), jnp.float32),
    )(kid3, tv3, sid3, key_table, time_sig_table, W1, b1r, W2, b2r, Wp, bpr)
